# baseline (speedup 1.0000x reference)
import functools

import jax
import jax.numpy as jnp
from jax import lax
from jax.experimental import pallas as pl
from jax.experimental.pallas import tpu as pltpu
from jax.experimental.pallas import tpu_sc as plsc

TIME_DIM = 16
NC = 2
NS = 16
NW = NC * NS
CHUNK = 256

INV_2PI = 0.15915494309189535
ROUND_MAGIC = 12582912.0
_COS_COEFFS = (
    -1.4531124,
    7.8001313,
    -26.404669,
    60.24213,
    -85.45666,
    64.93939,
    -19.739208,
    1.0,
)


def _cos_2pi(f):
    u = f * f
    p = jnp.float32(_COS_COEFFS[0])
    for c in _COS_COEFFS[1:]:
        p = p * u + jnp.float32(c)
    return p


def kernel(ts, entity, basis_freq, phase):
    B = ts.shape[0]
    b_per_w = B // NW
    f_per_w = b_per_w * TIME_DIM
    ent = entity.astype(jnp.int32)

    mesh = plsc.VectorSubcoreMesh(core_axis_name="c", subcore_axis_name="s")

    @functools.partial(
        pl.kernel,
        mesh=mesh,
        out_type=jax.ShapeDtypeStruct((B * TIME_DIM,), jnp.float32),
        scratch_types=[
            pltpu.VMEM((b_per_w,), jnp.int32),
            pltpu.VMEM((b_per_w,), jnp.float32),
            pltpu.VMEM((CHUNK, TIME_DIM), jnp.float32),
            pltpu.VMEM((CHUNK, TIME_DIM), jnp.float32),
            pltpu.VMEM((f_per_w,), jnp.float32),
            pltpu.SemaphoreType.DMA,
        ],
    )
    def sc_kernel(ts_hbm, ent_hbm, bf_hbm, ph_hbm, out_hbm,
                  idx_v, ts_v, bf_v, ph_v, out_v, sem):
        wid = lax.axis_index("s") * NC + lax.axis_index("c")
        base = wid * b_per_w
        pltpu.sync_copy(ent_hbm.at[pl.ds(base, b_per_w)], idx_v)
        pltpu.sync_copy(ts_hbm.at[pl.ds(base, b_per_w)], ts_v)

        for ch in range(b_per_w // CHUNK):
            off = ch * CHUNK

            def issue(g, carry):
                v = idx_v[pl.ds(off + g * 16, 16)]
                for j in range(16):
                    e = v[j]
                    i = g * 16 + j
                    pltpu.async_copy(bf_hbm.at[e], bf_v.at[i], sem)
                    pltpu.async_copy(ph_hbm.at[e], ph_v.at[i], sem)
                return carry

            lax.fori_loop(0, CHUNK // 16, issue, 0)
            pltpu.make_async_copy(bf_hbm.at[pl.ds(0, CHUNK)], bf_v, sem).wait()
            pltpu.make_async_copy(bf_hbm.at[pl.ds(0, CHUNK)], ph_v, sem).wait()

            def body(g, carry):
                tsv = ts_v[pl.ds(off + g * 16, 16)]
                for j in range(16):
                    i = g * 16 + j
                    t = tsv[j] * bf_v[i] + ph_v[i]
                    r = t * jnp.float32(INV_2PI)
                    n = (r + jnp.float32(ROUND_MAGIC)) - jnp.float32(ROUND_MAGIC)
                    out_v[pl.ds((off + i) * TIME_DIM, TIME_DIM)] = _cos_2pi(r - n)
                return carry

            lax.fori_loop(0, CHUNK // 16, body, 0)
        pltpu.sync_copy(out_v, out_hbm.at[pl.ds(wid * f_per_w, f_per_w)])

    return sc_kernel(ts, ent, basis_freq, phase).reshape(B, TIME_DIM)


# speedup vs baseline: 3.0130x; 3.0130x over previous
import functools

import jax
import jax.numpy as jnp
from jax import lax
from jax.experimental import pallas as pl
from jax.experimental.pallas import tpu as pltpu
from jax.experimental.pallas import tpu_sc as plsc

TIME_DIM = 16
NC = 2
NS = 16
NW = NC * NS
LANE = 128
GROUP = 16

INV_2PI = 0.15915494309189535
ROUND_MAGIC = 12582912.0
_COS_COEFFS = (
    -1.4531124,
    7.8001313,
    -26.404669,
    60.24213,
    -85.45666,
    64.93939,
    -19.739208,
    1.0,
)


def _cos_2pi(f):
    u = f * f
    p = jnp.float32(_COS_COEFFS[0])
    for c in _COS_COEFFS[1:]:
        p = p * u + jnp.float32(c)
    return p


def kernel(ts, entity, basis_freq, phase):
    B = ts.shape[0]
    V = basis_freq.shape[0]
    b_per_w = B // NW
    f_per_w = b_per_w * TIME_DIM
    n_groups = b_per_w // GROUP
    tail_start = (V // LANE) * LANE
    tail_w = V - tail_start
    last_full = tail_start - LANE
    ent = entity.astype(jnp.int32)
    bf_t = basis_freq.T
    ph_t = phase.T
    bf_tail = lax.slice(basis_freq, (tail_start, 0), (V, TIME_DIM))
    ph_tail = lax.slice(phase, (tail_start, 0), (V, TIME_DIM))

    mesh = plsc.VectorSubcoreMesh(core_axis_name="c", subcore_axis_name="s")

    @functools.partial(
        pl.kernel,
        mesh=mesh,
        out_type=jax.ShapeDtypeStruct((B * TIME_DIM,), jnp.float32),
        compiler_params=pltpu.CompilerParams(needs_layout_passes=False),
        scratch_types=[
            pltpu.VMEM((b_per_w,), jnp.int32),
            pltpu.VMEM((b_per_w,), jnp.float32),
            pltpu.VMEM((GROUP, TIME_DIM, LANE), jnp.float32),
            pltpu.VMEM((GROUP, TIME_DIM, LANE), jnp.float32),
            pltpu.VMEM((tail_w, TIME_DIM), jnp.float32),
            pltpu.VMEM((tail_w, TIME_DIM), jnp.float32),
            pltpu.VMEM((f_per_w,), jnp.float32),
            pltpu.SemaphoreType.DMA,
            pltpu.SemaphoreType.DMA,
        ],
    )
    def sc_kernel(ts_hbm, ent_hbm, bf_hbm, ph_hbm, bft_hbm, pht_hbm, out_hbm,
                  ent_v, ts_v, bf_b, ph_b, bf_tl, ph_tl, out_v, sem, sem2):
        wid = lax.axis_index("s") * NC + lax.axis_index("c")
        base = wid * b_per_w
        pltpu.sync_copy(ent_hbm.at[pl.ds(base, b_per_w)], ent_v)
        pltpu.sync_copy(ts_hbm.at[pl.ds(base, b_per_w)], ts_v)
        pltpu.sync_copy(bft_hbm, bf_tl)
        pltpu.sync_copy(pht_hbm, ph_tl)

        iota16 = lax.iota(jnp.int32, 16)
        zeros16 = jnp.zeros((16,), jnp.int32)

        def group_body(g, carry):
            ev = ent_v[pl.ds(g * GROUP, GROUP)]
            tsv = ts_v[pl.ds(g * GROUP, GROUP)]
            for half in range(2):
                copies = []
                for j in range(half * 8, half * 8 + 8):
                    e = ev[j]
                    lane = e & jnp.int32(LANE - 1)
                    s = jnp.minimum(e - lane, jnp.int32(last_full))
                    s = pl.multiple_of(s, LANE)
                    copies.append(pltpu.async_copy(
                        bf_hbm.at[:, pl.ds(s, LANE)], bf_b.at[j], sem))
                    copies.append(pltpu.async_copy(
                        ph_hbm.at[:, pl.ds(s, LANE)], ph_b.at[j], sem2))
                for cp in copies:
                    cp.wait()
            for j in range(GROUP):
                e = ev[j]
                lane = zeros16 + (e & jnp.int32(LANE - 1))
                slot = zeros16 + j
                bfr = plsc.load_gather(bf_b, [slot, iota16, lane])
                phr = plsc.load_gather(ph_b, [slot, iota16, lane])
                is_tail = e >= jnp.int32(tail_start)
                et = jnp.maximum(e - jnp.int32(tail_start), 0)
                bfr = jnp.where(is_tail, bf_tl[et], bfr)
                phr = jnp.where(is_tail, ph_tl[et], phr)
                t = tsv[j] * bfr + phr
                r = t * jnp.float32(INV_2PI)
                n = (r + jnp.float32(ROUND_MAGIC)) - jnp.float32(ROUND_MAGIC)
                i = g * GROUP + j
                out_v[pl.ds(i * TIME_DIM, TIME_DIM)] = _cos_2pi(r - n)
            return carry

        lax.fori_loop(0, n_groups, group_body, 0)
        pltpu.sync_copy(out_v, out_hbm.at[pl.ds(wid * f_per_w, f_per_w)])

    return sc_kernel(ts, ent, bf_t, ph_t, bf_tail, ph_tail).reshape(B, TIME_DIM)


# speedup vs baseline: 3.6155x; 1.2000x over previous
import functools

import jax
import jax.numpy as jnp
from jax import lax
from jax.experimental import pallas as pl
from jax.experimental.pallas import tpu as pltpu
from jax.experimental.pallas import tpu_sc as plsc

TIME_DIM = 16
NC = 2
NS = 16
NW = NC * NS
LANE = 128
HALF = 8

INV_2PI = 0.15915494309189535
ROUND_MAGIC = 12582912.0
_COS_COEFFS = (
    -1.4531124,
    7.8001313,
    -26.404669,
    60.24213,
    -85.45666,
    64.93939,
    -19.739208,
    1.0,
)


def _cos_2pi(f):
    u = f * f
    p = jnp.float32(_COS_COEFFS[0])
    for c in _COS_COEFFS[1:]:
        p = p * u + jnp.float32(c)
    return p


def kernel(ts, entity, basis_freq, phase):
    B = ts.shape[0]
    V = basis_freq.shape[0]
    b_per_w = B // NW
    f_per_w = b_per_w * TIME_DIM
    n_pairs = b_per_w // (2 * HALF)
    tail_start = (V // LANE) * LANE
    tail_w = V - tail_start
    last_full = tail_start - LANE
    ent = entity.astype(jnp.int32)
    bf_t = basis_freq.T
    ph_t = phase.T
    bf_tail = lax.slice(basis_freq, (tail_start, 0), (V, TIME_DIM))
    ph_tail = lax.slice(phase, (tail_start, 0), (V, TIME_DIM))

    mesh = plsc.VectorSubcoreMesh(core_axis_name="c", subcore_axis_name="s")

    @functools.partial(
        pl.kernel,
        mesh=mesh,
        out_type=jax.ShapeDtypeStruct((B * TIME_DIM,), jnp.float32),
        compiler_params=pltpu.CompilerParams(needs_layout_passes=False),
        scratch_types=[
            pltpu.VMEM((b_per_w,), jnp.int32),
            pltpu.VMEM((b_per_w,), jnp.float32),
            pltpu.VMEM((2 * HALF, TIME_DIM, LANE), jnp.float32),
            pltpu.VMEM((2 * HALF, TIME_DIM, LANE), jnp.float32),
            pltpu.VMEM((tail_w, TIME_DIM), jnp.float32),
            pltpu.VMEM((tail_w, TIME_DIM), jnp.float32),
            pltpu.VMEM((f_per_w,), jnp.float32),
            pltpu.SemaphoreType.DMA,
            pltpu.SemaphoreType.DMA,
        ],
    )
    def sc_kernel(ts_hbm, ent_hbm, bf_hbm, ph_hbm, bft_hbm, pht_hbm, out_hbm,
                  ent_v, ts_v, bf_b, ph_b, bf_tl, ph_tl, out_v, semA, semB):
        wid = lax.axis_index("s") * NC + lax.axis_index("c")
        base = wid * b_per_w
        pltpu.sync_copy(ent_hbm.at[pl.ds(base, b_per_w)], ent_v)
        pltpu.sync_copy(ts_hbm.at[pl.ds(base, b_per_w)], ts_v)
        pltpu.sync_copy(bft_hbm, bf_tl)
        pltpu.sync_copy(pht_hbm, ph_tl)

        iota16 = lax.iota(jnp.int32, 16)
        zeros16 = jnp.zeros((16,), jnp.int32)

        def issue_half(ev, lo, slot_base, sem):
            cps = []
            for idx, j in enumerate(range(lo, lo + HALF)):
                e = ev[j]
                lane = e & jnp.int32(LANE - 1)
                s = jnp.minimum(e - lane, jnp.int32(last_full))
                s = pl.multiple_of(s, LANE)
                cps.append(pltpu.async_copy(
                    bf_hbm.at[:, pl.ds(s, LANE)], bf_b.at[slot_base + idx], sem))
                cps.append(pltpu.async_copy(
                    ph_hbm.at[:, pl.ds(s, LANE)], ph_b.at[slot_base + idx], sem))
            return cps

        def drain_half(slot_base, sem):
            for idx in range(HALF):
                pltpu.make_async_copy(
                    bf_hbm.at[:, pl.ds(0, LANE)], bf_b.at[slot_base + idx],
                    sem).wait()
                pltpu.make_async_copy(
                    ph_hbm.at[:, pl.ds(0, LANE)], ph_b.at[slot_base + idx],
                    sem).wait()

        def compute_half(k, ev, tsv, lo, slot_base):
            for idx, j in enumerate(range(lo, lo + HALF)):
                e = ev[j]
                lane = zeros16 + (e & jnp.int32(LANE - 1))
                slot = zeros16 + (slot_base + idx)
                bfr = plsc.load_gather(bf_b, [slot, iota16, lane])
                phr = plsc.load_gather(ph_b, [slot, iota16, lane])
                is_tail = e >= jnp.int32(tail_start)
                et = jnp.maximum(e - jnp.int32(tail_start), 0)
                bfr = jnp.where(is_tail, bf_tl[et], bfr)
                phr = jnp.where(is_tail, ph_tl[et], phr)
                t = tsv[j] * bfr + phr
                r = t * jnp.float32(INV_2PI)
                n = (r + jnp.float32(ROUND_MAGIC)) - jnp.float32(ROUND_MAGIC)
                i = k * 2 * HALF + j
                out_v[pl.ds(i * TIME_DIM, TIME_DIM)] = _cos_2pi(r - n)

        ev0 = ent_v[pl.ds(0, 16)]
        issue_half(ev0, 0, 0, semA)

        def pair_body(k, carry):
            ev = ent_v[pl.ds(k * 2 * HALF, 16)]
            tsv = ts_v[pl.ds(k * 2 * HALF, 16)]
            cps1 = issue_half(ev, HALF, HALF, semB)
            drain_half(0, semA)
            compute_half(k, ev, tsv, 0, 0)

            @pl.when(k < n_pairs - 1)
            def _():
                evn = ent_v[pl.ds((k + 1) * 2 * HALF, 16)]
                issue_half(evn, 0, 0, semA)

            for cp in cps1:
                cp.wait()
            compute_half(k, ev, tsv, HALF, HALF)
            return carry

        lax.fori_loop(0, n_pairs, pair_body, 0)
        pltpu.sync_copy(out_v, out_hbm.at[pl.ds(wid * f_per_w, f_per_w)])

    return sc_kernel(ts, ent, bf_t, ph_t, bf_tail, ph_tail).reshape(B, TIME_DIM)
